# Initial kernel scaffold; baseline (speedup 1.0000x reference)
import jax
import jax.numpy as jnp
from jax.experimental import pallas as pl


def kernel(input_ids, p, s, tok_table, pos_table, lang_table, switch_emb, ln_gamma, ln_beta):
    raise NotImplementedError("write your pallas kernel here")


# speedup vs baseline: 1.8678x; 1.8678x over previous
import functools

import jax
import jax.numpy as jnp
from jax import lax
from jax.experimental import pallas as pl
from jax.experimental.pallas import tpu as pltpu
from jax.experimental.pallas import tpu_sc as plsc

B, T, D = 4, 2048, 768
V, K = 100000, 100
EPS = 1e-12

NC, NS = 2, 16
NW = NC * NS
N_TOK = B * T
ROWS_PER_W = N_TOK // NW
CHUNK = 64
NCHUNK = ROWS_PER_W // CHUNK

@functools.lru_cache(maxsize=1)
def _get_sc_gather():
    mesh = plsc.VectorSubcoreMesh(
        core_axis_name="c", subcore_axis_name="s",
        num_cores=NC, num_subcores=NS,
    )

    @functools.partial(
        pl.kernel,
        out_type=jax.ShapeDtypeStruct((N_TOK, D), jnp.float32),
        mesh=mesh,
        scratch_types=[
            pltpu.VMEM((NCHUNK, CHUNK), jnp.int32),
            pltpu.VMEM((2, CHUNK, D), jnp.float32),
            pltpu.SemaphoreType.DMA,
            pltpu.SemaphoreType.DMA,
        ],
    )
    def _sc_gather(tok_hbm, idx_hbm, out_hbm, idx_v, rows_v, sem0, sem1):
        wid = lax.axis_index("s") * NC + lax.axis_index("c")
        base = wid * ROWS_PER_W
        sems = (sem0, sem1)
        pltpu.sync_copy(idx_hbm.at[wid], idx_v)
        copies = [None] * NCHUNK
        copies[0] = pltpu.async_copy(
            tok_hbm.at[idx_v.at[0]], rows_v.at[0], sems[0])
        for c in range(NCHUNK):
            if c + 1 < NCHUNK:
                nb = (c + 1) % 2
                copies[c + 1] = pltpu.async_copy(
                    tok_hbm.at[idx_v.at[c + 1]], rows_v.at[nb], sems[nb]
                )
            copies[c].wait()
            pltpu.sync_copy(
                rows_v.at[c % 2], out_hbm.at[pl.ds(base + c * CHUNK, CHUNK)]
            )

    return _sc_gather


BT = 256
_N_BLK = N_TOK // BT
_POS_BLKS = T // BT


def _tc_body(gath_ref, p2_ref, lang2_ref, pos_ref, g_ref, b_ref, out_ref):
    x = gath_ref[...] + pos_ref[...]
    x = x + jnp.dot(p2_ref[...], lang2_ref[...],
                    preferred_element_type=jnp.float32)
    mu = jnp.mean(x, axis=1, keepdims=True)
    xc = x - mu
    var = jnp.mean(xc * xc, axis=1, keepdims=True)
    out_ref[...] = xc * lax.rsqrt(var + EPS) * g_ref[...] + b_ref[...]


_tc_fused = pl.pallas_call(
    _tc_body,
    grid=(_N_BLK,),
    in_specs=[
        pl.BlockSpec((BT, D), lambda i: (i, 0)),
        pl.BlockSpec((BT, K + 1), lambda i: (i, 0)),
        pl.BlockSpec((K + 1, D), lambda i: (0, 0)),
        pl.BlockSpec((BT, D), lambda i: (i % _POS_BLKS, 0)),
        pl.BlockSpec((1, D), lambda i: (0, 0)),
        pl.BlockSpec((1, D), lambda i: (0, 0)),
    ],
    out_specs=pl.BlockSpec((BT, D), lambda i: (i, 0)),
    out_shape=jax.ShapeDtypeStruct((N_TOK, D), jnp.float32),
)


def kernel(input_ids, p, s, tok_table, pos_table, lang_table, switch_emb,
           ln_gamma, ln_beta):
    ids = input_ids.astype(jnp.int32).reshape(NW, NCHUNK, CHUNK)
    gathered = _get_sc_gather()(tok_table, ids)
    p2 = jnp.concatenate(
        [p.reshape(N_TOK, K), s.reshape(N_TOK, 1)], axis=1)
    lang2 = jnp.concatenate([lang_table, switch_emb[None, :]], axis=0)
    out = _tc_fused(gathered, p2, lang2, pos_table,
                    ln_gamma[None, :], ln_beta[None, :])
    return out.reshape(B, T, D)


# speedup vs baseline: 2.1893x; 1.1721x over previous
import functools

import jax
import jax.numpy as jnp
from jax import lax
from jax.experimental import pallas as pl
from jax.experimental.pallas import tpu as pltpu
from jax.experimental.pallas import tpu_sc as plsc

B, T, D = 4, 2048, 768
V, K = 100000, 100
EPS = 1e-12

NC, NS = 2, 16
NW = NC * NS
N_TOK = B * T
ROWS_PER_W = N_TOK // NW
CHUNK = 64
NCHUNK = ROWS_PER_W // CHUNK

@functools.lru_cache(maxsize=1)
def _get_sc_gather():
    mesh = plsc.VectorSubcoreMesh(
        core_axis_name="c", subcore_axis_name="s",
        num_cores=NC, num_subcores=NS,
    )

    @functools.partial(
        pl.kernel,
        out_type=jax.ShapeDtypeStruct((N_TOK, D), jnp.float32),
        mesh=mesh,
        scratch_types=[
            pltpu.VMEM((NCHUNK, CHUNK), jnp.int32),
            pltpu.VMEM((2, CHUNK, D), jnp.float32),
            pltpu.SemaphoreType.DMA,
            pltpu.SemaphoreType.DMA,
        ],
    )
    def _sc_gather(tok_hbm, idx_hbm, out_hbm, idx_v, rows_v, sem0, sem1):
        wid = lax.axis_index("s") * NC + lax.axis_index("c")
        base = wid * ROWS_PER_W
        sems = (sem0, sem1)
        pltpu.sync_copy(idx_hbm.at[wid], idx_v)
        copies = [None] * NCHUNK
        copies[0] = pltpu.async_copy(
            tok_hbm.at[idx_v.at[0]], rows_v.at[0], sems[0])
        for c in range(NCHUNK):
            if c + 1 < NCHUNK:
                nb = (c + 1) % 2
                copies[c + 1] = pltpu.async_copy(
                    tok_hbm.at[idx_v.at[c + 1]], rows_v.at[nb], sems[nb]
                )
            copies[c].wait()
            pltpu.sync_copy(
                rows_v.at[c % 2], out_hbm.at[pl.ds(base + c * CHUNK, CHUNK)]
            )

    return _sc_gather


BT = 512
_T_BLKS = T // BT
_B_BLKS = B


def _tc_body(gath_ref, p_ref, s_ref, lang_ref, sw_ref, pos_ref,
             g_ref, b_ref, out_ref):
    x = gath_ref[...] + pos_ref[...]
    x = x + jnp.dot(p_ref[...], lang_ref[...],
                    preferred_element_type=jnp.float32)
    x = x + s_ref[...] * sw_ref[...]
    mu = jnp.mean(x, axis=1, keepdims=True)
    xc = x - mu
    var = jnp.mean(xc * xc, axis=1, keepdims=True)
    out_ref[...] = xc * lax.rsqrt(var + EPS) * g_ref[...] + b_ref[...]


_tc_fused = pl.pallas_call(
    _tc_body,
    grid=(_T_BLKS, _B_BLKS),
    in_specs=[
        pl.BlockSpec((BT, D), lambda i, j: (j * _T_BLKS + i, 0)),
        pl.BlockSpec((BT, K), lambda i, j: (j * _T_BLKS + i, 0)),
        pl.BlockSpec((BT, 1), lambda i, j: (j * _T_BLKS + i, 0)),
        pl.BlockSpec((K, D), lambda i, j: (0, 0)),
        pl.BlockSpec((1, D), lambda i, j: (0, 0)),
        pl.BlockSpec((BT, D), lambda i, j: (i, 0)),
        pl.BlockSpec((1, D), lambda i, j: (0, 0)),
        pl.BlockSpec((1, D), lambda i, j: (0, 0)),
    ],
    out_specs=pl.BlockSpec((BT, D), lambda i, j: (j * _T_BLKS + i, 0)),
    out_shape=jax.ShapeDtypeStruct((N_TOK, D), jnp.float32),
)


def kernel(input_ids, p, s, tok_table, pos_table, lang_table, switch_emb,
           ln_gamma, ln_beta):
    ids = input_ids.astype(jnp.int32).reshape(NW, NCHUNK, CHUNK)
    gathered = _get_sc_gather()(tok_table, ids)
    out = _tc_fused(gathered, p.reshape(N_TOK, K), s.reshape(N_TOK, 1),
                    lang_table, switch_emb[None, :], pos_table,
                    ln_gamma[None, :], ln_beta[None, :])
    return out.reshape(B, T, D)


# speedup vs baseline: 2.3100x; 1.0551x over previous
import functools

import jax
import jax.numpy as jnp
from jax import lax
from jax.experimental import pallas as pl
from jax.experimental.pallas import tpu as pltpu
from jax.experimental.pallas import tpu_sc as plsc

B, T, D = 4, 2048, 768
V, K = 100000, 100
EPS = 1e-12

NC, NS = 2, 16
NW = NC * NS
N_TOK = B * T
ROWS_PER_W = N_TOK // NW
CHUNK = 64
NCHUNK = ROWS_PER_W // CHUNK

@functools.lru_cache(maxsize=1)
def _get_sc_gather():
    mesh = plsc.VectorSubcoreMesh(
        core_axis_name="c", subcore_axis_name="s",
        num_cores=NC, num_subcores=NS,
    )

    @functools.partial(
        pl.kernel,
        out_type=jax.ShapeDtypeStruct((N_TOK, D), jnp.float32),
        mesh=mesh,
        scratch_types=[
            pltpu.VMEM((NCHUNK, CHUNK), jnp.int32),
            pltpu.VMEM((2, CHUNK, D), jnp.float32),
            pltpu.SemaphoreType.DMA,
            pltpu.SemaphoreType.DMA,
        ],
    )
    def _sc_gather(tok_hbm, idx_hbm, out_hbm, idx_v, rows_v, sem0, sem1):
        wid = lax.axis_index("s") * NC + lax.axis_index("c")
        base = wid * ROWS_PER_W
        sems = (sem0, sem1)
        pltpu.sync_copy(idx_hbm.at[wid], idx_v)
        copies = [None] * NCHUNK
        copies[0] = pltpu.async_copy(
            tok_hbm.at[idx_v.at[0]], rows_v.at[0], sems[0])
        for c in range(NCHUNK):
            if c + 1 < NCHUNK:
                nb = (c + 1) % 2
                copies[c + 1] = pltpu.async_copy(
                    tok_hbm.at[idx_v.at[c + 1]], rows_v.at[nb], sems[nb]
                )
            copies[c].wait()
            pltpu.sync_copy(
                rows_v.at[c % 2], out_hbm.at[pl.ds(base + c * CHUNK, CHUNK)]
            )

    return _sc_gather


BT = 1024
_T_BLKS = T // BT
_B_BLKS = B


def _tc_body(gath_ref, p_ref, s_ref, lang_ref, sw_ref, pos_ref,
             g_ref, b_ref, out_ref):
    x = gath_ref[...] + pos_ref[...]
    x = x + jnp.dot(p_ref[...], lang_ref[...],
                    preferred_element_type=jnp.float32)
    x = x + s_ref[...] * sw_ref[...]
    mu = jnp.mean(x, axis=1, keepdims=True)
    xc = x - mu
    var = jnp.mean(xc * xc, axis=1, keepdims=True)
    out_ref[...] = xc * lax.rsqrt(var + EPS) * g_ref[...] + b_ref[...]


_tc_fused = pl.pallas_call(
    _tc_body,
    grid=(_T_BLKS, _B_BLKS),
    in_specs=[
        pl.BlockSpec((BT, D), lambda i, j: (j * _T_BLKS + i, 0)),
        pl.BlockSpec((BT, K), lambda i, j: (j * _T_BLKS + i, 0)),
        pl.BlockSpec((BT, 1), lambda i, j: (j * _T_BLKS + i, 0)),
        pl.BlockSpec((K, D), lambda i, j: (0, 0)),
        pl.BlockSpec((1, D), lambda i, j: (0, 0)),
        pl.BlockSpec((BT, D), lambda i, j: (i, 0)),
        pl.BlockSpec((1, D), lambda i, j: (0, 0)),
        pl.BlockSpec((1, D), lambda i, j: (0, 0)),
    ],
    out_specs=pl.BlockSpec((BT, D), lambda i, j: (j * _T_BLKS + i, 0)),
    out_shape=jax.ShapeDtypeStruct((N_TOK, D), jnp.float32),
)


def kernel(input_ids, p, s, tok_table, pos_table, lang_table, switch_emb,
           ln_gamma, ln_beta):
    ids = input_ids.astype(jnp.int32).reshape(NW, NCHUNK, CHUNK)
    gathered = _get_sc_gather()(tok_table, ids)
    out = _tc_fused(gathered, p.reshape(N_TOK, K), s.reshape(N_TOK, 1),
                    lang_table, switch_emb[None, :], pos_table,
                    ln_gamma[None, :], ln_beta[None, :])
    return out.reshape(B, T, D)
